# baseline (speedup 1.0000x reference)
import functools

import jax
import jax.numpy as jnp
from jax import lax
from jax.experimental import pallas as pl
from jax.experimental.pallas import tpu as pltpu
from jax.experimental.pallas import tpu_sc as plsc

N_NODES = 10000
N_EDGES = 320000
D = 128

NC = 2
NS = 16
NW = NC * NS

E_PER_TILE = N_EDGES // NW
E_CHK = 40
N_RING = 5
N_CHK = E_PER_TILE // E_CHK
N_PAD = 10240
ROWS_PER_TILE = N_PAD // NS

@functools.lru_cache(maxsize=1)
def _make_sc_aggregate():
    mesh = plsc.VectorSubcoreMesh(core_axis_name="c", subcore_axis_name="s")

    @functools.partial(
        pl.kernel,
        mesh=mesh,
        out_type=jax.ShapeDtypeStruct((NC * N_PAD, D), jnp.float32),
        scratch_types=[
            pltpu.VMEM((E_PER_TILE,), jnp.int32),
            pltpu.VMEM((E_PER_TILE,), jnp.int32),
            *[pltpu.VMEM((E_CHK, D), jnp.float32) for _ in range(N_RING)],
            pltpu.VMEM_SHARED((N_PAD, D), jnp.float32),
            *[pltpu.SemaphoreType.DMA for _ in range(N_RING)],
        ],
    )
    def _sc_aggregate(edges_hbm, x_hbm, zeros_hbm, out_hbm,
                      sidx, didx, *rest):
        rows = rest[:N_RING]
        acc = rest[N_RING]
        sems = rest[N_RING + 1:]
        c = lax.axis_index("c")
        s = lax.axis_index("s")
        tile = s * NC + c
        row0 = s * ROWS_PER_TILE
        edge0 = tile * E_PER_TILE

        pltpu.sync_copy(edges_hbm.at[pl.ds(edge0, E_PER_TILE)], sidx)
        pltpu.sync_copy(edges_hbm.at[pl.ds(N_EDGES + edge0, E_PER_TILE)], didx)
        pltpu.sync_copy(zeros_hbm, acc.at[pl.ds(row0, ROWS_PER_TILE)])
        plsc.subcore_barrier()

        def gather_start(chk, buf, sem):
            pltpu.async_copy(
                x_hbm.at[sidx.at[pl.ds(chk * E_CHK, E_CHK)]], buf, sem)

        def gather_wait(chk, buf, sem):
            pltpu.make_async_copy(
                x_hbm.at[sidx.at[pl.ds(chk * E_CHK, E_CHK)]], buf, sem).wait()

        def scatter(chk, buf):
            pltpu.sync_copy(buf, acc.at[didx.at[pl.ds(chk * E_CHK, E_CHK)]],
                            add=True)

        for b in range(N_RING):
            gather_start(b, rows[b], sems[b])

        def step(j, carry):
            a = N_RING * j
            for b in range(N_RING):
                gather_wait(a + b, rows[b], sems[b])
                scatter(a + b, rows[b])

                @pl.when(j < N_CHK // N_RING - 1)
                def _(b=b):
                    gather_start(a + b + N_RING, rows[b], sems[b])

            return carry

        lax.fori_loop(0, N_CHK // N_RING, step, 0)

        plsc.subcore_barrier()
        pltpu.sync_copy(acc.at[pl.ds(row0, ROWS_PER_TILE)],
                        out_hbm.at[pl.ds(c * N_PAD + row0, ROWS_PER_TILE)])

    return _sc_aggregate


def _tc_root_body(x_ref, wr_ref, b_ref, o_ref):
    o_ref[...] = jnp.dot(x_ref[...], wr_ref[...],
                         preferred_element_type=jnp.float32) + b_ref[...]


def _tc_combine_body(d_ref, p0_ref, p1_ref, wn_ref, o_ref):
    agg = p0_ref[...] + p1_ref[...]
    o = d_ref[...] + jnp.dot(agg, wn_ref[...],
                             preferred_element_type=jnp.float32)
    o_ref[...] = jnp.maximum(o, 0.0)


_BLK = 1280
_NBLK = N_PAD // _BLK


def kernel(x, edge_index, W_root, W_nbr, b):
    edges = edge_index.astype(jnp.int32).reshape(-1)
    zeros = jnp.zeros((ROWS_PER_TILE, D), jnp.float32)

    partials = _make_sc_aggregate()(edges, x, zeros)

    dense1 = pl.pallas_call(
        _tc_root_body,
        grid=(_NBLK,),
        in_specs=[
            pl.BlockSpec((_BLK, D), lambda i: (i, 0)),
            pl.BlockSpec((D, D), lambda i: (0, 0)),
            pl.BlockSpec((1, D), lambda i: (0, 0)),
        ],
        out_specs=pl.BlockSpec((_BLK, D), lambda i: (i, 0)),
        out_shape=jax.ShapeDtypeStruct((N_NODES, D), jnp.float32),
    )(x, W_root, b.reshape(1, D))

    out = pl.pallas_call(
        _tc_combine_body,
        grid=(_NBLK,),
        in_specs=[
            pl.BlockSpec((_BLK, D), lambda i: (i, 0)),
            pl.BlockSpec((_BLK, D), lambda i: (i, 0)),
            pl.BlockSpec((_BLK, D), lambda i: (i + _NBLK, 0)),
            pl.BlockSpec((D, D), lambda i: (0, 0)),
        ],
        out_specs=pl.BlockSpec((_BLK, D), lambda i: (i, 0)),
        out_shape=jax.ShapeDtypeStruct((N_NODES, D), jnp.float32),
    )(dense1, partials, partials, W_nbr)
    return out


# speedup vs baseline: 1.0153x; 1.0153x over previous
import functools

import jax
import jax.numpy as jnp
from jax import lax
from jax.experimental import pallas as pl
from jax.experimental.pallas import tpu as pltpu
from jax.experimental.pallas import tpu_sc as plsc

N_NODES = 10000
N_EDGES = 320000
D = 128

NC = 2
NS = 16
NW = NC * NS

E_PER_TILE = N_EDGES // NW
E_CHK = 40
N_RING = 5
N_CHK = E_PER_TILE // E_CHK
N_PAD = 10240
ROWS_PER_TILE = N_PAD // NS

@functools.lru_cache(maxsize=1)
def _make_sc_aggregate():
    mesh = plsc.VectorSubcoreMesh(core_axis_name="c", subcore_axis_name="s")

    @functools.partial(
        pl.kernel,
        mesh=mesh,
        out_type=jax.ShapeDtypeStruct((NC * N_PAD, D), jnp.float32),
        scratch_types=[
            pltpu.VMEM((E_PER_TILE,), jnp.int32),
            pltpu.VMEM((E_PER_TILE,), jnp.int32),
            *[pltpu.VMEM((E_CHK, D), jnp.float32) for _ in range(N_RING)],
            pltpu.VMEM_SHARED((N_PAD, D), jnp.float32),
            *[pltpu.SemaphoreType.DMA for _ in range(N_RING)],
        ],
    )
    def _sc_aggregate(edges_hbm, x_hbm, zeros_hbm, out_hbm,
                      sidx, didx, *rest):
        rows = rest[:N_RING]
        acc = rest[N_RING]
        sems = rest[N_RING + 1:]
        c = lax.axis_index("c")
        s = lax.axis_index("s")
        tile = s * NC + c
        row0 = s * ROWS_PER_TILE
        edge0 = tile * E_PER_TILE

        pltpu.sync_copy(edges_hbm.at[pl.ds(edge0, E_PER_TILE)], sidx)
        pltpu.sync_copy(edges_hbm.at[pl.ds(N_EDGES + edge0, E_PER_TILE)], didx)
        pltpu.sync_copy(zeros_hbm, acc.at[pl.ds(row0, ROWS_PER_TILE)])
        plsc.subcore_barrier()

        def gather_start(chk, buf, sem):
            pltpu.async_copy(
                x_hbm.at[sidx.at[pl.ds(chk * E_CHK, E_CHK)]], buf, sem)

        def gather_wait(chk, buf, sem):
            pltpu.make_async_copy(
                x_hbm.at[sidx.at[pl.ds(chk * E_CHK, E_CHK)]], buf, sem).wait()

        def scatter(chk, buf):
            pltpu.sync_copy(buf, acc.at[didx.at[pl.ds(chk * E_CHK, E_CHK)]],
                            add=True)

        for b in range(N_RING):
            gather_start(b, rows[b], sems[b])

        def step(j, carry):
            a = N_RING * j
            for b in range(N_RING):
                gather_wait(a + b, rows[b], sems[b])
                scatter(a + b, rows[b])

                @pl.when(j < N_CHK // N_RING - 1)
                def _(b=b):
                    gather_start(a + b + N_RING, rows[b], sems[b])

            return carry

        lax.fori_loop(0, N_CHK // N_RING, step, 0)

        plsc.subcore_barrier()
        pltpu.sync_copy(acc.at[pl.ds(row0, ROWS_PER_TILE)],
                        out_hbm.at[pl.ds(c * N_PAD + row0, ROWS_PER_TILE)])

    return _sc_aggregate


def _tc_body(x_ref, p0_ref, p1_ref, wr_ref, wn_ref, b_ref, o_ref):
    agg = p0_ref[...] + p1_ref[...]
    o = jnp.dot(x_ref[...], wr_ref[...], preferred_element_type=jnp.float32)
    o += jnp.dot(agg, wn_ref[...], preferred_element_type=jnp.float32)
    o += b_ref[...]
    o_ref[...] = jnp.maximum(o, 0.0)


_BLK = 1280
_NBLK = N_PAD // _BLK


def kernel(x, edge_index, W_root, W_nbr, b):
    edges = edge_index.astype(jnp.int32).reshape(-1)
    zeros = jnp.zeros((ROWS_PER_TILE, D), jnp.float32)

    partials = _make_sc_aggregate()(edges, x, zeros)

    out = pl.pallas_call(
        _tc_body,
        grid=(_NBLK,),
        in_specs=[
            pl.BlockSpec((_BLK, D), lambda i: (i, 0)),
            pl.BlockSpec((_BLK, D), lambda i: (i, 0)),
            pl.BlockSpec((_BLK, D), lambda i: (i + _NBLK, 0)),
            pl.BlockSpec((D, D), lambda i: (0, 0)),
            pl.BlockSpec((D, D), lambda i: (0, 0)),
            pl.BlockSpec((1, D), lambda i: (0, 0)),
        ],
        out_specs=pl.BlockSpec((_BLK, D), lambda i: (i, 0)),
        out_shape=jax.ShapeDtypeStruct((N_NODES, D), jnp.float32),
    )(x, partials, partials, W_root, W_nbr, b.reshape(1, D))
    return out
